# baseline (speedup 1.0000x reference)
import functools

import jax
import jax.numpy as jnp
from jax import lax
from jax.experimental import pallas as pl
from jax.experimental.pallas import tpu as pltpu
from jax.experimental.pallas import tpu_sc as plsc

_B = 16384
_D = 128
_NCORES = 2
_NSUB = 16
_NW = _NCORES * _NSUB

_NCHUNK = 4
_NB = _B // _NCHUNK
_BM = 2048


def _make_sc_body(nb, chunk_base):
    bpw = nb // _NW

    def body(uid_hbm, qid_hbm, user_t, qdiff_t, qtab_t, qdisc_t,
             u_out, d_out, m_out, disc_out,
             uid_v, qid_v, rows_v, disc_v, sem, dsem):
        wid = lax.axis_index("s") * _NCORES + lax.axis_index("c")
        base = wid * bpw
        pltpu.sync_copy(uid_hbm.at[pl.ds(chunk_base + base, bpw)], uid_v)
        pltpu.sync_copy(qid_hbm.at[pl.ds(chunk_base + base, bpw)], qid_v)
        gd = pltpu.async_copy(qdisc_t.at[qid_v], disc_v, dsem)
        pltpu.async_copy(user_t.at[uid_v], rows_v, sem).wait()
        pltpu.sync_copy(rows_v, u_out.at[pl.ds(base, bpw)])
        pltpu.async_copy(qdiff_t.at[qid_v], rows_v, sem).wait()
        pltpu.sync_copy(rows_v, d_out.at[pl.ds(base, bpw)])
        pltpu.async_copy(qtab_t.at[qid_v], rows_v, sem).wait()
        pltpu.sync_copy(rows_v, m_out.at[pl.ds(base, bpw)])
        gd.wait()
        pltpu.sync_copy(disc_v, disc_out.at[pl.ds(base, bpw)])

    return body


@functools.cache
def _sc_gather(nb, chunk_base):
    bpw = nb // _NW
    return pl.kernel(
        _make_sc_body(nb, chunk_base),
        out_type=[
            jax.ShapeDtypeStruct((nb, _D), jnp.float32),
            jax.ShapeDtypeStruct((nb, _D), jnp.float32),
            jax.ShapeDtypeStruct((nb, _D), jnp.float32),
            jax.ShapeDtypeStruct((nb,), jnp.float32),
        ],
        mesh=plsc.VectorSubcoreMesh(core_axis_name="c", subcore_axis_name="s",
                                    num_cores=_NCORES, num_subcores=_NSUB),
        scratch_types=[
            pltpu.VMEM((bpw,), jnp.int32),
            pltpu.VMEM((bpw,), jnp.int32),
            pltpu.VMEM((bpw, _D), jnp.float32),
            pltpu.VMEM((bpw,), jnp.float32),
            pltpu.SemaphoreType.DMA,
            pltpu.SemaphoreType.DMA,
        ],
    )


def _sigmoid(x):
    return 0.5 * jnp.tanh(0.5 * x) + 0.5


def _mlp_body(u_ref, d_ref, m_ref, disc_ref, w1_ref, b1_ref, w2_ref, b2_ref,
              w3t_ref, b3_ref, out_ref):
    u = _sigmoid(u_ref[...])
    d = _sigmoid(d_ref[...])
    disc = _sigmoid(disc_ref[...].T) * 10.0
    x = disc * (u - d) * m_ref[...]
    h = _sigmoid(
        jnp.dot(x, w1_ref[...], preferred_element_type=jnp.float32) + b1_ref[...])
    h = _sigmoid(
        jnp.dot(h, w2_ref[...], preferred_element_type=jnp.float32) + b2_ref[...])
    o = jnp.sum(h * w3t_ref[...], axis=-1, keepdims=True) + b3_ref[...]
    out_ref[...] = _sigmoid(o).T


def _tc_mlp(u_rows, d_rows, m_rows, disc, W1, b1r, W2, b2r, w3t, b3r):
    nb = u_rows.shape[0]
    grid = nb // _BM
    row_spec = pl.BlockSpec((_BM, _D), lambda i: (i, 0))
    rowvec_spec = pl.BlockSpec((1, _BM), lambda i: (0, i))
    full = lambda shape: pl.BlockSpec(shape, lambda i: (0,) * len(shape))
    return pl.pallas_call(
        _mlp_body,
        grid=(grid,),
        in_specs=[
            row_spec, row_spec, row_spec, rowvec_spec,
            full((128, 512)), full((1, 512)),
            full((512, 256)), full((1, 256)),
            full((1, 256)), full((1, 1)),
        ],
        out_specs=rowvec_spec,
        out_shape=jax.ShapeDtypeStruct((1, nb), jnp.float32),
        compiler_params=pltpu.CompilerParams(
            dimension_semantics=("arbitrary",)),
    )(u_rows, d_rows, m_rows, disc.reshape(1, nb), W1, b1r, W2, b2r, w3t, b3r)


@jax.jit
def _ncd_forward(uid, qid, q_table, user_table, q_diff_table, q_disc_table,
                 W1, b1, W2, b2, W3, b3):
    qdisc_1d = q_disc_table.reshape(-1)
    b1r = b1.reshape(1, -1)
    b2r = b2.reshape(1, -1)
    w3t = W3.reshape(1, -1)
    b3r = b3.reshape(1, 1)
    gathered = []
    for c in range(_NCHUNK):
        gathered.append(_sc_gather(_NB, c * _NB)(
            uid, qid, user_table, q_diff_table, q_table, qdisc_1d))
    outs = []
    for c in range(_NCHUNK):
        u_rows, d_rows, m_rows, disc = gathered[c]
        outs.append(_tc_mlp(u_rows, d_rows, m_rows, disc,
                            W1, b1r, W2, b2r, w3t, b3r))
    return jnp.concatenate(outs, axis=1).reshape(-1)


def kernel(user_id, question_id, q_table, user_table, q_diff_table,
           q_disc_table, W1, b1, W2, b2, W3, b3):
    uid = user_id.astype(jnp.int32)
    qid = question_id.astype(jnp.int32)
    return _ncd_forward(uid, qid, q_table, user_table, q_diff_table,
                        q_disc_table, W1, b1, W2, b2, W3, b3)


# speedup vs baseline: 1.1827x; 1.1827x over previous
import functools

import jax
import jax.numpy as jnp
from jax import lax
from jax.experimental import pallas as pl
from jax.experimental.pallas import tpu as pltpu
from jax.experimental.pallas import tpu_sc as plsc

_B = 16384
_D = 128
_NCORES = 2
_NSUB = 16
_NW = _NCORES * _NSUB

_NCHUNK = 1
_NB = _B // _NCHUNK
_BM = 4096


def _make_sc_body(nb, chunk_base):
    bpw = nb // _NW

    def body(uid_hbm, qid_hbm, user_t, qdiff_t, qtab_t, qdisc_t,
             u_out, d_out, m_out, disc_out,
             uid_v, qid_v, rows_v, disc_v, sem, dsem):
        wid = lax.axis_index("s") * _NCORES + lax.axis_index("c")
        base = wid * bpw
        pltpu.sync_copy(uid_hbm.at[pl.ds(chunk_base + base, bpw)], uid_v)
        pltpu.sync_copy(qid_hbm.at[pl.ds(chunk_base + base, bpw)], qid_v)
        gd = pltpu.async_copy(qdisc_t.at[qid_v], disc_v, dsem)
        pltpu.async_copy(user_t.at[uid_v], rows_v, sem).wait()
        pltpu.sync_copy(rows_v, u_out.at[pl.ds(base, bpw)])
        pltpu.async_copy(qdiff_t.at[qid_v], rows_v, sem).wait()
        pltpu.sync_copy(rows_v, d_out.at[pl.ds(base, bpw)])
        pltpu.async_copy(qtab_t.at[qid_v], rows_v, sem).wait()
        pltpu.sync_copy(rows_v, m_out.at[pl.ds(base, bpw)])
        gd.wait()
        pltpu.sync_copy(disc_v, disc_out.at[pl.ds(base, bpw)])

    return body


@functools.cache
def _sc_gather(nb, chunk_base):
    bpw = nb // _NW
    return pl.kernel(
        _make_sc_body(nb, chunk_base),
        out_type=[
            jax.ShapeDtypeStruct((nb, _D), jnp.float32),
            jax.ShapeDtypeStruct((nb, _D), jnp.float32),
            jax.ShapeDtypeStruct((nb, _D), jnp.float32),
            jax.ShapeDtypeStruct((nb,), jnp.float32),
        ],
        mesh=plsc.VectorSubcoreMesh(core_axis_name="c", subcore_axis_name="s",
                                    num_cores=_NCORES, num_subcores=_NSUB),
        scratch_types=[
            pltpu.VMEM((bpw,), jnp.int32),
            pltpu.VMEM((bpw,), jnp.int32),
            pltpu.VMEM((bpw, _D), jnp.float32),
            pltpu.VMEM((bpw,), jnp.float32),
            pltpu.SemaphoreType.DMA,
            pltpu.SemaphoreType.DMA,
        ],
    )


def _mlp_body(u_ref, d_ref, m_ref, disc_ref, w1_ref, b1_ref, w2_ref, b2_ref,
              w3t_ref, b3_ref, out_ref):
    ut = jnp.tanh(u_ref[...] * 0.5)
    dt = jnp.tanh(d_ref[...] * 0.5)
    dct = jnp.tanh(disc_ref[...].T * 0.5) + 1.0
    x = (((ut - dt) * m_ref[...]) * dct).astype(jnp.bfloat16)
    t1 = jnp.tanh(
        jnp.dot(x, w1_ref[...], preferred_element_type=jnp.float32) + b1_ref[...])
    t2 = jnp.tanh(
        jnp.dot(t1.astype(jnp.bfloat16), w2_ref[...],
                preferred_element_type=jnp.float32) + b2_ref[...])
    o = jnp.sum(t2 * w3t_ref[...], axis=-1, keepdims=True) + b3_ref[...]
    out_ref[...] = (0.5 * jnp.tanh(o) + 0.5).T


def _tc_mlp(u_rows, d_rows, m_rows, disc, W1, b1r, W2, b2r, w3t, b3r):
    nb = u_rows.shape[0]
    grid = nb // _BM
    row_spec = pl.BlockSpec((_BM, _D), lambda i: (i, 0))
    rowvec_spec = pl.BlockSpec((1, _BM), lambda i: (0, i))
    full = lambda shape: pl.BlockSpec(shape, lambda i: (0,) * len(shape))
    return pl.pallas_call(
        _mlp_body,
        grid=(grid,),
        in_specs=[
            row_spec, row_spec, row_spec, rowvec_spec,
            full((128, 512)), full((1, 512)),
            full((512, 256)), full((1, 256)),
            full((1, 256)), full((1, 1)),
        ],
        out_specs=rowvec_spec,
        out_shape=jax.ShapeDtypeStruct((1, nb), jnp.float32),
        compiler_params=pltpu.CompilerParams(
            dimension_semantics=("arbitrary",)),
    )(u_rows, d_rows, m_rows, disc.reshape(1, nb), W1, b1r, W2, b2r, w3t, b3r)


@jax.jit
def _ncd_forward(uid, qid, q_table, user_table, q_diff_table, q_disc_table,
                 W1, b1, W2, b2, W3, b3):
    qdisc_1d = q_disc_table.reshape(-1)
    b1r = (0.5 * b1).reshape(1, -1)
    b2r = (0.25 * jnp.sum(W2, axis=0) + 0.5 * b2).reshape(1, -1)
    w3t = (0.25 * W3).reshape(1, -1)
    b3r = (0.25 * jnp.sum(W3) + 0.5 * b3).reshape(1, 1)
    W1 = (1.25 * W1).astype(jnp.bfloat16)
    W2 = (0.25 * W2).astype(jnp.bfloat16)
    gathered = []
    for c in range(_NCHUNK):
        gathered.append(_sc_gather(_NB, c * _NB)(
            uid, qid, user_table, q_diff_table, q_table, qdisc_1d))
    outs = []
    for c in range(_NCHUNK):
        u_rows, d_rows, m_rows, disc = gathered[c]
        outs.append(_tc_mlp(u_rows, d_rows, m_rows, disc,
                            W1, b1r, W2, b2r, w3t, b3r))
    return jnp.concatenate(outs, axis=1).reshape(-1)


def kernel(user_id, question_id, q_table, user_table, q_diff_table,
           q_disc_table, W1, b1, W2, b2, W3, b3):
    uid = user_id.astype(jnp.int32)
    qid = question_id.astype(jnp.int32)
    return _ncd_forward(uid, qid, q_table, user_table, q_diff_table,
                        q_disc_table, W1, b1, W2, b2, W3, b3)


# speedup vs baseline: 1.2355x; 1.0446x over previous
import functools

import jax
import jax.numpy as jnp
from jax import lax
from jax.experimental import pallas as pl
from jax.experimental.pallas import tpu as pltpu
from jax.experimental.pallas import tpu_sc as plsc

_B = 16384
_D = 128
_NCORES = 2
_NSUB = 16
_NW = _NCORES * _NSUB
_BPW = _B // _NW
_CH = 128
_NCH = _BPW // _CH

_BM = 4096


def _chunk_compute(u_buf, d_buf, m_buf):
    def row_body(r, _):
        for j in range(_D // 16):
            sl = pl.ds(j * 16, 16)
            u = u_buf[r, sl]
            d = d_buf[r, sl]
            m = m_buf[r, sl]
            eu = jnp.exp(-u)
            ed = jnp.exp(-d)
            num = ed - eu
            den = (1.0 + eu) * (1.0 + ed)
            u_buf[r, sl] = m * (num / den)
        return 0

    lax.fori_loop(0, _CH, row_body, 0)


def _sc_body(uid_hbm, qid_hbm, user_t, qdiff_t, qtab_t, qdisc_t,
             ym_out, disc_out,
             uid_v, qid_v, u0, d0, m0, u1, d1, m1, disc_v,
             gsem0, gsem1, ssem, dsem):
    wid = lax.axis_index("s") * _NCORES + lax.axis_index("c")
    base = wid * _BPW
    pltpu.sync_copy(uid_hbm.at[pl.ds(base, _BPW)], uid_v)
    pltpu.sync_copy(qid_hbm.at[pl.ds(base, _BPW)], qid_v)
    gd = pltpu.async_copy(qdisc_t.at[qid_v], disc_v, dsem)

    bufs = ((u0, d0, m0), (u1, d1, m1))
    gsems = (gsem0, gsem1)

    def issue_gathers(k):
        s = k % 2
        ub, db, mb = bufs[s]
        sem = gsems[s]
        sl = pl.ds(k * _CH, _CH)
        return (pltpu.async_copy(user_t.at[uid_v.at[sl]], ub, sem),
                pltpu.async_copy(qdiff_t.at[qid_v.at[sl]], db, sem),
                pltpu.async_copy(qtab_t.at[qid_v.at[sl]], mb, sem))

    gathers = [None, None]
    scatters = [None, None]
    gathers[0] = issue_gathers(0)
    for k in range(_NCH):
        s = k % 2
        if k + 1 < _NCH:
            if scatters[(k + 1) % 2] is not None:
                scatters[(k + 1) % 2].wait()
                scatters[(k + 1) % 2] = None
            gathers[(k + 1) % 2] = issue_gathers(k + 1)
        for g in gathers[s]:
            g.wait()
        ub, db, mb = bufs[s]
        _chunk_compute(ub, db, mb)
        scatters[s] = pltpu.async_copy(
            ub, ym_out.at[pl.ds(base + k * _CH, _CH)], ssem)
    for sc in scatters:
        if sc is not None:
            sc.wait()
    gd.wait()
    pltpu.sync_copy(disc_v, disc_out.at[pl.ds(base, _BPW)])


@functools.cache
def _sc_gather():
    return pl.kernel(
        _sc_body,
        out_type=[
            jax.ShapeDtypeStruct((_B, _D), jnp.float32),
            jax.ShapeDtypeStruct((_B,), jnp.float32),
        ],
        mesh=plsc.VectorSubcoreMesh(core_axis_name="c", subcore_axis_name="s",
                                    num_cores=_NCORES, num_subcores=_NSUB),
        scratch_types=[
            pltpu.VMEM((_BPW,), jnp.int32),
            pltpu.VMEM((_BPW,), jnp.int32),
            pltpu.VMEM((_CH, _D), jnp.float32),
            pltpu.VMEM((_CH, _D), jnp.float32),
            pltpu.VMEM((_CH, _D), jnp.float32),
            pltpu.VMEM((_CH, _D), jnp.float32),
            pltpu.VMEM((_CH, _D), jnp.float32),
            pltpu.VMEM((_CH, _D), jnp.float32),
            pltpu.VMEM((_BPW,), jnp.float32),
            pltpu.SemaphoreType.DMA,
            pltpu.SemaphoreType.DMA,
            pltpu.SemaphoreType.DMA,
            pltpu.SemaphoreType.DMA,
        ],
    )


def _mlp_body(ym_ref, disc_ref, w1_ref, b1_ref, w2_ref, b2_ref,
              w3t_ref, b3_ref, out_ref):
    dct = jnp.tanh(disc_ref[...].T * 0.5) + 1.0
    x = (ym_ref[...] * dct).astype(jnp.bfloat16)
    t1 = jnp.tanh(
        jnp.dot(x, w1_ref[...], preferred_element_type=jnp.float32) + b1_ref[...])
    t2 = jnp.tanh(
        jnp.dot(t1.astype(jnp.bfloat16), w2_ref[...],
                preferred_element_type=jnp.float32) + b2_ref[...])
    o = jnp.sum(t2 * w3t_ref[...], axis=-1, keepdims=True) + b3_ref[...]
    out_ref[...] = (0.5 * jnp.tanh(o) + 0.5).T


@jax.jit
def _ncd_forward(uid, qid, q_table, user_table, q_diff_table, q_disc_table,
                 W1, b1, W2, b2, W3, b3):
    qdisc_1d = q_disc_table.reshape(-1)
    b1r = (0.5 * b1).reshape(1, -1)
    b2r = (0.25 * jnp.sum(W2, axis=0) + 0.5 * b2).reshape(1, -1)
    w3t = (0.25 * W3).reshape(1, -1)
    b3r = (0.25 * jnp.sum(W3) + 0.5 * b3).reshape(1, 1)
    W1q = (2.5 * W1).astype(jnp.bfloat16)
    W2q = (0.25 * W2).astype(jnp.bfloat16)

    ym, disc = _sc_gather()(
        uid, qid, user_table, q_diff_table, q_table, qdisc_1d)

    grid = _B // _BM
    row_spec = pl.BlockSpec((_BM, _D), lambda i: (i, 0))
    rowvec_spec = pl.BlockSpec((1, _BM), lambda i: (0, i))
    full = lambda shape: pl.BlockSpec(shape, lambda i: (0,) * len(shape))
    out = pl.pallas_call(
        _mlp_body,
        grid=(grid,),
        in_specs=[
            row_spec, rowvec_spec,
            full((128, 512)), full((1, 512)),
            full((512, 256)), full((1, 256)),
            full((1, 256)), full((1, 1)),
        ],
        out_specs=rowvec_spec,
        out_shape=jax.ShapeDtypeStruct((1, _B), jnp.float32),
        compiler_params=pltpu.CompilerParams(
            dimension_semantics=("arbitrary",)),
    )(ym, disc.reshape(1, _B), W1q, b1r, W2q, b2r, w3t, b3r)
    return out.reshape(-1)


def kernel(user_id, question_id, q_table, user_table, q_diff_table,
           q_disc_table, W1, b1, W2, b2, W3, b3):
    uid = user_id.astype(jnp.int32)
    qid = question_id.astype(jnp.int32)
    return _ncd_forward(uid, qid, q_table, user_table, q_diff_table,
                        q_disc_table, W1, b1, W2, b2, W3, b3)
